# baseline (speedup 1.0000x reference)
import functools

import jax
import jax.numpy as jnp
from jax import lax
from jax.experimental import pallas as pl
from jax.experimental.pallas import tpu as pltpu
from jax.experimental.pallas import tpu_sc as plsc

EPS = 1e-3


def _xy_kernel(a_ref, w1_ref, c1_ref, w2_ref, c2_ref, x_ref, y_ref):
    xb = jnp.dot(a_ref[...], w1_ref[...], preferred_element_type=jnp.float32)
    xb = xb + c1_ref[...]
    x_ref[...] = xb
    y_ref[...] = (
        jnp.dot(xb, w2_ref[...], preferred_element_type=jnp.float32) + c2_ref[...]
    )


def _tc_xy(atom, a1, c1, a2, c2):
    n, d = atom.shape
    u = a1.shape[1]
    blk = 2000
    assert n % blk == 0
    return pl.pallas_call(
        _xy_kernel,
        grid=(n // blk,),
        in_specs=[
            pl.BlockSpec((blk, d), lambda i: (i, 0)),
            pl.BlockSpec((d, u), lambda i: (0, 0)),
            pl.BlockSpec((1, u), lambda i: (0, 0)),
            pl.BlockSpec((u, u), lambda i: (0, 0)),
            pl.BlockSpec((1, u), lambda i: (0, 0)),
        ],
        out_specs=[
            pl.BlockSpec((blk, u), lambda i: (i, 0)),
            pl.BlockSpec((blk, u), lambda i: (i, 0)),
        ],
        out_shape=[
            jax.ShapeDtypeStruct((n, u), jnp.float32),
            jax.ShapeDtypeStruct((n, u), jnp.float32),
        ],
    )(atom, a1, c1, a2, c2)


def _bonds_kernel(bf_ref, w_ref, c_ref, o_ref):
    v = (
        jnp.dot(bf_ref[...], w_ref[...], preferred_element_type=jnp.float32)
        + c_ref[...]
    )
    bits = jax.lax.bitcast_convert_type(
        v.astype(jnp.bfloat16).astype(jnp.float32), jnp.uint32)
    halves = []
    n128 = bits.shape[1] // 128
    for e in range(n128):
        a = bits[:, 128 * e:128 * e + 64]
        b = bits[:, 128 * e + 64:128 * e + 128]
        halves.append((a >> 16) | (b & jnp.uint32(0xFFFF0000)))
    o_ref[...] = jax.lax.bitcast_convert_type(
        jnp.concatenate(halves, axis=1), jnp.int32)


def _tc_bonds(bf2, w3big, c3big):
    m2, k2 = bf2.shape
    uo = w3big.shape[1]
    blk = 4000
    assert m2 % blk == 0
    return pl.pallas_call(
        _bonds_kernel,
        grid=(m2 // blk,),
        in_specs=[
            pl.BlockSpec((blk, k2), lambda i: (i, 0)),
            pl.BlockSpec((k2, uo), lambda i: (0, 0)),
            pl.BlockSpec((1, uo), lambda i: (0, 0)),
        ],
        out_specs=pl.BlockSpec((blk, uo // 2), lambda i: (i, 0)),
        out_shape=jax.ShapeDtypeStruct((m2, uo // 2), jnp.int32),
    )(bf2, w3big, c3big)


_CHUNK = 80


def _make_sc_edge_kernel(n, e, u):
    nc, ns = 2, 16
    nw = nc * ns
    ew = e // nw
    nchunks = ew // _CHUNK
    assert ew * nw == e and nchunks * _CHUNK == ew
    assert nchunks % 4 == 1 and nchunks >= 9
    rpt = 640
    npad = rpt * ns
    assert npad >= n and n % _CHUNK == 0 and u == 128
    mesh = plsc.VectorSubcoreMesh(core_axis_name="c", subcore_axis_name="s")
    ewc = e // nc

    @functools.partial(
        pl.kernel,
        mesh=mesh,
        out_type=jax.ShapeDtypeStruct((nc, n, u), jnp.float32),
        scratch_types=[
            pltpu.VMEM((4, 2, _CHUNK), jnp.int32),
            pltpu.VMEM((2, _CHUNK, u), jnp.float32),
            pltpu.VMEM((2, _CHUNK // 2, u), jnp.int32),
            pltpu.VMEM_SHARED((npad, u), jnp.float32),
            pltpu.SemaphoreType.DMA,
            pltpu.SemaphoreType.DMA,
            pltpu.SemaphoreType.DMA,
            pltpu.SemaphoreType.DMA,
            pltpu.SemaphoreType.DMA,
            pltpu.SemaphoreType.DMA,
        ],
    )
    def sc_kernel(y_hbm, bonds_hbm, src_hbm, dst_hbm, out_hbm,
                  idx_v, rows_v, bonds_v, acc,
                  i0, i1, i2, i3, dm0, dm1):
        c = lax.axis_index("c")
        s = lax.axis_index("s")
        isem = (i0, i1, i2, i3)
        dsem = (dm0, dm1)

        def zero_row(r, carry):
            for j in range(u // 16):
                rows_v[0, r, pl.ds(j * 16, 16)] = jnp.zeros((16,), jnp.float32)
            return carry

        lax.fori_loop(0, _CHUNK, zero_row, 0)
        for z in range(rpt // _CHUNK):
            pltpu.sync_copy(rows_v.at[0],
                            acc.at[pl.ds(s * rpt + z * _CHUNK, _CHUNK)])
        plsc.subcore_barrier()

        def issue_idx(kk, j):
            lbase = s * ew + kk * _CHUNK
            pltpu.async_copy(src_hbm.at[pl.ds(c * ewc + lbase, _CHUNK)],
                             idx_v.at[j, 0], isem[j])
            pltpu.async_copy(dst_hbm.at[pl.ds(c * ewc + lbase, _CHUNK)],
                             idx_v.at[j, 1], isem[j])

        def issue_data(kk, j, b):
            pltpu.make_async_copy(src_hbm.at[pl.ds(0, _CHUNK)],
                                  idx_v.at[j, 0], isem[j]).wait()
            pltpu.make_async_copy(dst_hbm.at[pl.ds(0, _CHUNK)],
                                  idx_v.at[j, 1], isem[j]).wait()
            lbase = s * ew + kk * _CHUNK
            pltpu.async_copy(y_hbm.at[idx_v.at[j, 0]], rows_v.at[b], dsem[b])
            lbase2 = c * (ewc // 2) + s * (ew // 2) + kk * (_CHUNK // 2)
            pltpu.async_copy(
                bonds_hbm.at[pl.ds(lbase2, _CHUNK // 2)],
                bonds_v.at[b], dsem[b])

        def finish(j, b):
            pltpu.make_async_copy(y_hbm.at[pl.ds(0, _CHUNK)],
                                  rows_v.at[b], dsem[b]).wait()
            pltpu.make_async_copy(bonds_hbm.at[pl.ds(0, _CHUNK // 2)],
                                  bonds_v.at[b], dsem[b]).wait()

            def mul_rows(r4, inner):
                for rr in range(4):
                    r = r4 * 4 + rr
                    p = r4 * 2 + rr // 2
                    cb = 64 * (rr % 2)
                    for jw in range(u // 32):
                        w = bonds_v[b, p, pl.ds(cb + jw * 16, 16)]
                        flo = jax.lax.bitcast_convert_type(
                            w * jnp.int32(65536), jnp.float32)
                        fhi = jax.lax.bitcast_convert_type(
                            jnp.bitwise_and(w, jnp.int32(-65536)), jnp.float32)
                        slo = pl.ds(jw * 32, 16)
                        shi = pl.ds(jw * 32 + 16, 16)
                        rows_v[b, r, slo] = rows_v[b, r, slo] * flo
                        rows_v[b, r, shi] = rows_v[b, r, shi] * fhi
                return inner

            lax.fori_loop(0, _CHUNK // 4, mul_rows, 0)
            pltpu.sync_copy(rows_v.at[b], acc.at[idx_v.at[j, 1]], add=True)

        def step(kk, j, b, do_idx=True, do_data=True):
            if do_idx:
                issue_idx(kk + 2, (j + 2) % 4)
            if do_data:
                issue_data(kk + 1, (j + 1) % 4, (b + 1) % 2)
            finish(j, b)

        issue_idx(0, 0)
        issue_idx(1, 1)
        issue_data(0, 0, 0)

        for j in range(4):
            step(j, j, j % 2)

        def quad(t, carry):
            for j in range(4):
                step(4 * t + j, j, j % 2)
            return carry

        lax.fori_loop(1, (nchunks - 5) // 4, quad, 0)

        for j in range(4):
            kk = nchunks - 5 + j
            step(kk, j, kk % 2,
                 do_idx=(kk + 2 < nchunks), do_data=(kk + 1 < nchunks))
        step(nchunks - 1, 0, (nchunks - 1) % 2, do_idx=False, do_data=False)
        plsc.subcore_barrier()

        for z in range(rpt // _CHUNK):
            row0 = s * rpt + z * _CHUNK

            @pl.when(row0 + _CHUNK <= n)
            def _():
                pltpu.sync_copy(acc.at[pl.ds(row0, _CHUNK)],
                                out_hbm.at[c, pl.ds(row0, _CHUNK)])

    return sc_kernel


def _final_kernel(x_ref, p0_ref, p1_ref, o_ref):
    o_ref[...] = x_ref[...] + p0_ref[0] + p1_ref[0]


def _tc_final(x, partial):
    n, u = x.shape
    blk = 400
    return pl.pallas_call(
        _final_kernel,
        grid=(n // blk,),
        in_specs=[
            pl.BlockSpec((blk, u), lambda i: (i, 0)),
            pl.BlockSpec((1, blk, u), lambda i: (0, i, 0)),
            pl.BlockSpec((1, blk, u), lambda i: (1, i, 0)),
        ],
        out_specs=pl.BlockSpec((blk, u), lambda i: (i, 0)),
        out_shape=jax.ShapeDtypeStruct((n, u), jnp.float32),
    )(x, partial, partial)


def kernel(atom_features, bond_features, bond_pairs,
           W1, b1, g1, be1, W2, b2, g2, be2, W3, b3, g3, be3):
    n, d = atom_features.shape
    e, de = bond_features.shape
    u = W1.shape[1]

    scale = (1.0 + EPS) ** -0.5
    s1 = g1 * scale
    a1 = W1 * s1[None, :]
    c1 = (b1 * s1 + be1)[None, :]
    s2 = g2 * scale
    a2 = W2 * s2[None, :]
    c2 = (b2 * s2 + be2)[None, :]
    s3 = g3 * scale
    a3 = W3 * s3[None, :]
    c3 = b3 * s3 + be3

    x, y = _tc_xy(atom_features, a1, c1, a2, c2)

    pi = [0] * u
    for g in range(u // 32):
        for l16 in range(16):
            pi[32 * g + l16 - 16 * g] = 0
    lo_cols = []
    hi_cols = []
    for g in range(u // 32):
        lo_cols += list(range(32 * g, 32 * g + 16))
        hi_cols += list(range(32 * g + 16, 32 * g + 32))
    pi = jnp.array(lo_cols + hi_cols, dtype=jnp.int32)
    a3p = a3[:, pi]
    c3p = c3[pi]
    w3big = jnp.kron(jnp.eye(2, dtype=jnp.float32), a3p)
    c3big = jnp.tile(c3p, 2)[None, :]
    bf2 = bond_features.reshape(e // 2, 2 * de)
    bonds_t = _tc_bonds(bf2, w3big, c3big)

    src = bond_pairs[:, 1]
    dst = bond_pairs[:, 0]
    partial = _make_sc_edge_kernel(n, e, u)(y, bonds_t, src, dst)
    return _tc_final(x, partial)


# speedup vs baseline: 1.3179x; 1.3179x over previous
import functools

import jax
import jax.numpy as jnp
from jax import lax
from jax.experimental import pallas as pl
from jax.experimental.pallas import tpu as pltpu
from jax.experimental.pallas import tpu_sc as plsc

EPS = 1e-3


def _xy_kernel(a_ref, w1_ref, c1_ref, w2_ref, c2_ref, x_ref, y_ref):
    xb = jnp.dot(a_ref[...], w1_ref[...], preferred_element_type=jnp.float32)
    xb = xb + c1_ref[...]
    x_ref[...] = xb
    y_ref[...] = (
        jnp.dot(xb, w2_ref[...], preferred_element_type=jnp.float32) + c2_ref[...]
    )


def _tc_xy(atom, a1, c1, a2, c2):
    n, d = atom.shape
    u = a1.shape[1]
    blk = 2000
    assert n % blk == 0
    return pl.pallas_call(
        _xy_kernel,
        grid=(n // blk,),
        in_specs=[
            pl.BlockSpec((blk, d), lambda i: (i, 0)),
            pl.BlockSpec((d, u), lambda i: (0, 0)),
            pl.BlockSpec((1, u), lambda i: (0, 0)),
            pl.BlockSpec((u, u), lambda i: (0, 0)),
            pl.BlockSpec((1, u), lambda i: (0, 0)),
        ],
        out_specs=[
            pl.BlockSpec((blk, u), lambda i: (i, 0)),
            pl.BlockSpec((blk, u), lambda i: (i, 0)),
        ],
        out_shape=[
            jax.ShapeDtypeStruct((n, u), jnp.float32),
            jax.ShapeDtypeStruct((n, u), jnp.float32),
        ],
    )(atom, a1, c1, a2, c2)


def _bonds_kernel(bf_ref, w_ref, c_ref, o_ref):
    o_ref[...] = (
        jnp.dot(bf_ref[...], w_ref[...], preferred_element_type=jnp.float32)
        + c_ref[...]
    )


def _tc_bonds(bf, a3, c3):
    m, k = bf.shape
    uo = a3.shape[1]
    blk = 8000
    assert m % blk == 0
    return pl.pallas_call(
        _bonds_kernel,
        grid=(m // blk,),
        in_specs=[
            pl.BlockSpec((blk, k), lambda i: (i, 0)),
            pl.BlockSpec((k, uo), lambda i: (0, 0)),
            pl.BlockSpec((1, uo), lambda i: (0, 0)),
        ],
        out_specs=pl.BlockSpec((blk, uo), lambda i: (i, 0)),
        out_shape=jax.ShapeDtypeStruct((m, uo), jnp.float32),
    )(bf, a3, c3)


_CHUNK = 80


def _make_sc_edge_kernel(n, e, u):
    nc, ns = 2, 16
    nw = nc * ns
    ew = e // nw
    nchunks = ew // _CHUNK
    assert ew * nw == e and nchunks * _CHUNK == ew
    assert nchunks % 4 == 1 and nchunks >= 9
    rpt = 640
    npad = rpt * ns
    assert npad >= n and n % _CHUNK == 0 and u == 128
    mesh = plsc.VectorSubcoreMesh(core_axis_name="c", subcore_axis_name="s")
    ewc = e // nc

    @functools.partial(
        pl.kernel,
        mesh=mesh,
        out_type=jax.ShapeDtypeStruct((nc, n, u), jnp.float32),
        scratch_types=[
            pltpu.VMEM((4, 2, _CHUNK), jnp.int32),
            pltpu.VMEM((2, _CHUNK, u), jnp.float32),
            pltpu.VMEM((2, _CHUNK, u), jnp.float32),
            pltpu.VMEM_SHARED((npad, u), jnp.float32),
            pltpu.SemaphoreType.DMA,
            pltpu.SemaphoreType.DMA,
            pltpu.SemaphoreType.DMA,
            pltpu.SemaphoreType.DMA,
            pltpu.SemaphoreType.DMA,
            pltpu.SemaphoreType.DMA,
        ],
    )
    def sc_kernel(y_hbm, bonds_hbm, src_hbm, dst_hbm, out_hbm,
                  idx_v, rows_v, bonds_v, acc,
                  i0, i1, i2, i3, dm0, dm1):
        c = lax.axis_index("c")
        s = lax.axis_index("s")
        isem = (i0, i1, i2, i3)
        dsem = (dm0, dm1)

        def zero_row(r, carry):
            for j in range(u // 16):
                rows_v[0, r, pl.ds(j * 16, 16)] = jnp.zeros((16,), jnp.float32)
            return carry

        lax.fori_loop(0, _CHUNK, zero_row, 0)
        for z in range(rpt // _CHUNK):
            pltpu.sync_copy(rows_v.at[0],
                            acc.at[pl.ds(s * rpt + z * _CHUNK, _CHUNK)])
        plsc.subcore_barrier()

        def issue_idx(kk, j):
            lbase = s * ew + kk * _CHUNK
            pltpu.async_copy(src_hbm.at[pl.ds(c * ewc + lbase, _CHUNK)],
                             idx_v.at[j, 0], isem[j])
            pltpu.async_copy(dst_hbm.at[pl.ds(c * ewc + lbase, _CHUNK)],
                             idx_v.at[j, 1], isem[j])

        def issue_data(kk, j, b):
            pltpu.make_async_copy(src_hbm.at[pl.ds(0, _CHUNK)],
                                  idx_v.at[j, 0], isem[j]).wait()
            pltpu.make_async_copy(dst_hbm.at[pl.ds(0, _CHUNK)],
                                  idx_v.at[j, 1], isem[j]).wait()
            lbase = s * ew + kk * _CHUNK
            pltpu.async_copy(y_hbm.at[idx_v.at[j, 0]], rows_v.at[b], dsem[b])
            pltpu.async_copy(bonds_hbm.at[pl.ds(c * ewc + lbase, _CHUNK)],
                             bonds_v.at[b], dsem[b])

        def finish(j, b):
            pltpu.make_async_copy(y_hbm.at[pl.ds(0, _CHUNK)],
                                  rows_v.at[b], dsem[b]).wait()
            pltpu.make_async_copy(bonds_hbm.at[pl.ds(0, _CHUNK)],
                                  bonds_v.at[b], dsem[b]).wait()

            def mul_rows(r4, inner):
                for rr in range(4):
                    r = r4 * 4 + rr
                    for jj in range(u // 16):
                        sl = pl.ds(jj * 16, 16)
                        rows_v[b, r, sl] = rows_v[b, r, sl] * bonds_v[b, r, sl]
                return inner

            lax.fori_loop(0, _CHUNK // 4, mul_rows, 0)
            pltpu.sync_copy(rows_v.at[b], acc.at[idx_v.at[j, 1]], add=True)

        def step(kk, j, b, do_idx=True, do_data=True):
            if do_idx:
                issue_idx(kk + 2, (j + 2) % 4)
            if do_data:
                issue_data(kk + 1, (j + 1) % 4, (b + 1) % 2)
            finish(j, b)

        issue_idx(0, 0)
        issue_idx(1, 1)
        issue_data(0, 0, 0)

        for j in range(4):
            step(j, j, j % 2)

        def quad(t, carry):
            for j in range(4):
                step(4 * t + j, j, j % 2)
            return carry

        lax.fori_loop(1, (nchunks - 5) // 4, quad, 0)

        for j in range(4):
            kk = nchunks - 5 + j
            step(kk, j, kk % 2,
                 do_idx=(kk + 2 < nchunks), do_data=(kk + 1 < nchunks))
        step(nchunks - 1, 0, (nchunks - 1) % 2, do_idx=False, do_data=False)
        plsc.subcore_barrier()

        for z in range(rpt // _CHUNK):
            row0 = s * rpt + z * _CHUNK

            @pl.when(row0 + _CHUNK <= n)
            def _():
                pltpu.sync_copy(acc.at[pl.ds(row0, _CHUNK)],
                                out_hbm.at[c, pl.ds(row0, _CHUNK)])

    return sc_kernel


def _final_kernel(x_ref, p0_ref, p1_ref, o_ref):
    o_ref[...] = x_ref[...] + p0_ref[0] + p1_ref[0]


def _tc_final(x, partial):
    n, u = x.shape
    blk = 400
    return pl.pallas_call(
        _final_kernel,
        grid=(n // blk,),
        in_specs=[
            pl.BlockSpec((blk, u), lambda i: (i, 0)),
            pl.BlockSpec((1, blk, u), lambda i: (0, i, 0)),
            pl.BlockSpec((1, blk, u), lambda i: (1, i, 0)),
        ],
        out_specs=pl.BlockSpec((blk, u), lambda i: (i, 0)),
        out_shape=jax.ShapeDtypeStruct((n, u), jnp.float32),
    )(x, partial, partial)


def kernel(atom_features, bond_features, bond_pairs,
           W1, b1, g1, be1, W2, b2, g2, be2, W3, b3, g3, be3):
    n, d = atom_features.shape
    e, de = bond_features.shape
    u = W1.shape[1]

    scale = (1.0 + EPS) ** -0.5
    s1 = g1 * scale
    a1 = W1 * s1[None, :]
    c1 = (b1 * s1 + be1)[None, :]
    s2 = g2 * scale
    a2 = W2 * s2[None, :]
    c2 = (b2 * s2 + be2)[None, :]
    s3 = g3 * scale
    a3 = W3 * s3[None, :]
    c3 = b3 * s3 + be3

    x, y = _tc_xy(atom_features, a1, c1, a2, c2)

    bonds_t = _tc_bonds(bond_features, a3, c3[None, :])

    src = bond_pairs[:, 1]
    dst = bond_pairs[:, 0]
    partial = _make_sc_edge_kernel(n, e, u)(y, bonds_t, src, dst)
    return _tc_final(x, partial)
